# Initial kernel scaffold; baseline (speedup 1.0000x reference)
import jax
import jax.numpy as jnp
from jax.experimental import pallas as pl


def kernel(x_ingredient, x_direction, cond, edge_co_occurs_with, edge_used_in, edge_contains, edge_pairs_with, edge_follows, params):
    raise NotImplementedError("write your pallas kernel here")


# speedup vs baseline: 45.2268x; 45.2268x over previous
import functools

import jax
import jax.numpy as jnp
from jax import lax
from jax.experimental import pallas as pl
from jax.experimental.pallas import tpu as pltpu
from jax.experimental.pallas import tpu_sc as plsc

N = 50000
NP = 51200
E = 800000
H = 64
HH = 32
F = 128
C = 16
CB = 2048
NBLK = NP // CB
PB = NP // 16
CH = 400
NCH = E // 16 // CH
NSUB = PB // CH
F32 = jnp.float32


def _prep_body(xi_ref, xd_ref, wi_ref, bi_ref, wd_ref, bd_ref, atti_ref,
               attd_ref, xio_ref, xdo_ref, svi_ref, svd_ref, ami_ref, amd_ref):
    k = pl.program_id(0)
    yi = jnp.dot(xi_ref[...], wi_ref[...],
                 preferred_element_type=F32) + bi_ref[0:1, :]
    yd = jnp.dot(xd_ref[...], wd_ref[...],
                 preferred_element_type=F32) + bd_ref[0:1, :]
    xio_ref[0] = yi[:, :HH]
    xio_ref[1] = yi[:, HH:]
    xdo_ref[0] = yd[:, :HH]
    xdo_ref[1] = yd[:, HH:]
    svi = lax.dot_general(atti_ref[...], yi, (((1,), (1,)), ((), ())),
                          preferred_element_type=F32)
    svd = lax.dot_general(attd_ref[...], yd, (((1,), (1,)), ((), ())),
                          preferred_element_type=F32)
    svi_ref[...] = svi
    svd_ref[...] = svd
    mi = jnp.broadcast_to(jnp.max(svi, axis=1, keepdims=True), (8, 128))
    md = jnp.broadcast_to(jnp.max(svd, axis=1, keepdims=True), (8, 128))

    @pl.when(k == 0)
    def _():
        ami_ref[...] = mi
        amd_ref[...] = md

    @pl.when(k != 0)
    def _():
        ami_ref[...] = jnp.maximum(ami_ref[...], mi)
        amd_ref[...] = jnp.maximum(amd_ref[...], md)


def _sc_body(e0, e1, e2, e3, e4, xic, xdc, svi, svd, a5, z2d, z1d,
             out_edge,
             acc, den, s_tab, d_tab,
             jv, iv, jv2, sg, dg, wv, rows, av,
             sem_s, sem_d, sem_r):
    cid = lax.axis_index("c")
    sid = lax.axis_index("s")
    segbase = sid * PB
    xoff = cid * NP

    es = [e0, e1, e2, e3, e4]
    plan = [(svi, 0, svd, 0, xic),
            (svi, 1, svd, 1, xic),
            (svd, 2, svd, 3, xdc),
            (svd, 4, svd, 5, xdc),
            (svd, 6, svi, 2, xdc)]

    for r in range(5):
        s_ref, s_row, d_ref, d_row, x_ref = plan[r]
        e_ref = es[r]
        pltpu.sync_copy(s_ref.at[s_row, pl.ds(segbase, PB)],
                        s_tab.at[pl.ds(segbase, PB)])
        pltpu.sync_copy(d_ref.at[d_row, pl.ds(segbase, PB)],
                        d_tab.at[pl.ds(segbase, PB)])
        pltpu.sync_copy(z2d, acc.at[pl.ds(segbase, PB)])
        pltpu.sync_copy(z1d, den.at[pl.ds(segbase, PB)])
        pltpu.sync_copy(a5.at[pl.ds(r * 16, 16)], av)
        plsc.subcore_barrier()

        def chunk(c, carry):
            base = sid * (E // 16) + c * CH
            pltpu.sync_copy(e_ref.at[pl.ds(base, CH)], jv)
            pltpu.sync_copy(e_ref.at[pl.ds(E + base, CH)], iv)
            cp_s = pltpu.async_copy(s_tab.at[jv], sg, sem_s)
            cp_d = pltpu.async_copy(d_tab.at[iv], dg, sem_d)

            def adj(g, _):
                sl = pl.ds(g * 16, 16)
                jv2[sl] = jv[sl] + xoff
                return 0
            lax.fori_loop(0, CH // 16, adj, 0)
            cp_r = pltpu.async_copy(x_ref.at[jv2], rows, sem_r)
            cp_s.wait()
            cp_d.wait()
            avv = av[...]

            def wloop(g, _):
                sl = pl.ds(g * 16, 16)
                t = sg[sl] + dg[sl]
                al = jnp.maximum(t, 0.2 * t)
                bb = avv + dg[sl]
                b = jnp.maximum(bb, 0.2 * bb)
                wv[sl] = jnp.exp(al - b)
                return 0
            lax.fori_loop(0, CH // 16, wloop, 0)
            cp_r.wait()

            def mloop(g, _):
                w16 = wv[pl.ds(g * 16, 16)]
                for lane in range(16):
                    e = g * 16 + lane
                    w = w16[lane]
                    rows[e, pl.ds(0, 16)] = rows[e, pl.ds(0, 16)] * w
                    rows[e, pl.ds(16, 16)] = rows[e, pl.ds(16, 16)] * w
                return 0
            lax.fori_loop(0, CH // 16, mloop, 0)
            pltpu.sync_copy(rows, acc.at[iv], add=True)
            pltpu.sync_copy(wv, den.at[iv], add=True)
            return carry
        lax.fori_loop(0, NCH, chunk, 0)
        plsc.subcore_barrier()

        for sub in range(NSUB):
            off = segbase + sub * CH
            pltpu.sync_copy(acc.at[pl.ds(off, CH)], rows)
            pltpu.sync_copy(den.at[pl.ds(off, CH)], sg)

            def dloop(g, _):
                inv16 = 1.0 / (sg[pl.ds(g * 16, 16)] + 1e-16)
                for lane in range(16):
                    e = g * 16 + lane
                    inv = inv16[lane]
                    rows[e, pl.ds(0, 16)] = jnp.maximum(
                        rows[e, pl.ds(0, 16)] * inv, 0.0)
                    rows[e, pl.ds(16, 16)] = jnp.maximum(
                        rows[e, pl.ds(16, 16)] * inv, 0.0)
                return 0
            lax.fori_loop(0, CH // 16, dloop, 0)
            pltpu.sync_copy(rows, out_edge.at[r, cid, pl.ds(off, CH)])


def _post_body(lo_ref, hi_ref, kw_ref, kb_ref, s1_ref, t_ref):
    k = pl.program_id(0)
    ids = lax.broadcasted_iota(jnp.int32, (PB, 1), 0) + k * PB
    mask = ids < N
    cs = []
    ct = []
    for r in range(5):
        al = lo_ref[r, 0]
        ah = hi_ref[r, 0]
        s1 = jnp.concatenate([jnp.sum(al, axis=0), jnp.sum(ah, axis=0)])
        pre = (jnp.dot(al, kw_ref[:HH, :], preferred_element_type=F32)
               + jnp.dot(ah, kw_ref[HH:, :], preferred_element_type=F32)
               + kb_ref[0:1, :])
        th = jnp.where(mask, jnp.tanh(pre), 0.0)
        cs.append(s1[None, :])
        ct.append(jnp.sum(th, axis=0)[None, :])
    cs = jnp.concatenate(cs, axis=0)
    ct = jnp.concatenate(ct, axis=0)

    @pl.when(k == 0)
    def _():
        s1_ref[...] = cs
        t_ref[...] = ct

    @pl.when(k != 0)
    def _():
        s1_ref[...] = s1_ref[...] + cs
        t_ref[...] = t_ref[...] + ct


def _tail_body(s1_ref, t_ref, q_ref, cond_ref, wmua_ref, wmub_ref, wmuc_ref,
               bmu_ref, wlva_ref, wlvb_ref, wlvc_ref, blv_ref, w3_ref, b3_ref,
               w2_ref, b2_ref, eps_ref, recon_ref, mu_ref, lv_ref):
    s1 = s1_ref[...]
    kmat = t_ref[...] * (1.0 / N)
    scores = jnp.sum(kmat * q_ref[0:1, :], axis=1, keepdims=True)
    sc4 = scores[0:4]
    m = jnp.max(sc4)
    e = jnp.exp(sc4 - m)
    attn = e / jnp.sum(e)
    mean_dir = jnp.sum(attn * s1[0:4], axis=0, keepdims=True) * (1.0 / N)
    mean_ing = s1[4:5] * (1.0 / N)
    mi8 = jnp.broadcast_to(mean_ing, (8, H))
    md8 = jnp.broadcast_to(mean_dir, (8, H))
    cond = cond_ref[...]

    def head(wa, wb, wc, b):
        return (jnp.dot(mi8, wa[...], preferred_element_type=F32)
                + jnp.dot(md8, wb[...], preferred_element_type=F32)
                + jnp.dot(cond, wc[...], preferred_element_type=F32)
                + b[0:1, :])
    mu = head(wmua_ref, wmub_ref, wmuc_ref, bmu_ref)
    lv = head(wlva_ref, wlvb_ref, wlvc_ref, blv_ref)
    std = jnp.exp(0.5 * lv)
    z = mu + eps_ref[...] * std
    h = jnp.maximum(jnp.dot(z, w3_ref[...], preferred_element_type=F32)
                    + b3_ref[0:1, :], 0.0)
    recon = jnp.tanh(jnp.dot(h, w2_ref[...], preferred_element_type=F32)
                     + b2_ref[0:1, :])
    recon_ref[...] = recon
    mu_ref[...] = mu
    lv_ref[...] = lv


def _b8(v, w):
    return jnp.broadcast_to(v[None, :], (8, w)).astype(F32)


def kernel(x_ingredient, x_direction, cond, edge_co_occurs_with, edge_used_in,
           edge_contains, edge_pairs_with, edge_follows, params):
    p = params
    atti = jnp.stack([p["att_src_co_occurs_with"], p["att_src_used_in"],
                      p["att_dst_contains"]]
                     + [jnp.zeros((H,), F32)] * 5)
    attd = jnp.stack([p["att_dst_co_occurs_with"], p["att_dst_used_in"],
                      p["att_src_pairs_with"], p["att_dst_pairs_with"],
                      p["att_src_follows"], p["att_dst_follows"],
                      p["att_src_contains"], jnp.zeros((H,), F32)])

    xic, xdc, svi, svd, ami, amd = pl.pallas_call(
        _prep_body,
        grid=(NBLK,),
        in_specs=[
            pl.BlockSpec((CB, F), lambda k: (k, 0)),
            pl.BlockSpec((CB, F), lambda k: (k, 0)),
            pl.BlockSpec((F, H), lambda k: (0, 0)),
            pl.BlockSpec((8, H), lambda k: (0, 0)),
            pl.BlockSpec((F, H), lambda k: (0, 0)),
            pl.BlockSpec((8, H), lambda k: (0, 0)),
            pl.BlockSpec((8, H), lambda k: (0, 0)),
            pl.BlockSpec((8, H), lambda k: (0, 0)),
        ],
        out_specs=[
            pl.BlockSpec((2, CB, HH), lambda k: (0, k, 0)),
            pl.BlockSpec((2, CB, HH), lambda k: (0, k, 0)),
            pl.BlockSpec((8, CB), lambda k: (0, k)),
            pl.BlockSpec((8, CB), lambda k: (0, k)),
            pl.BlockSpec((8, 128), lambda k: (0, 0)),
            pl.BlockSpec((8, 128), lambda k: (0, 0)),
        ],
        out_shape=[
            jax.ShapeDtypeStruct((2, NP, HH), F32),
            jax.ShapeDtypeStruct((2, NP, HH), F32),
            jax.ShapeDtypeStruct((8, NP), F32),
            jax.ShapeDtypeStruct((8, NP), F32),
            jax.ShapeDtypeStruct((8, 128), F32),
            jax.ShapeDtypeStruct((8, 128), F32),
        ],
    )(jnp.pad(x_ingredient, ((0, NP - N), (0, 0))),
      jnp.pad(x_direction, ((0, NP - N), (0, 0))), p["proj_W_ingredient"],
      _b8(p["proj_b_ingredient"], H), p["proj_W_direction"],
      _b8(p["proj_b_direction"], H), atti, attd)

    a5 = jnp.broadcast_to(
        jnp.stack([ami[0, 0], ami[1, 0], amd[2, 0], amd[4, 0],
                   amd[6, 0]])[:, None], (5, 16)).reshape(80)
    z2d = jnp.zeros((PB, HH), F32)
    z1d = jnp.zeros((PB,), F32)

    sc_edge = pl.kernel(
        _sc_body,
        out_type=[jax.ShapeDtypeStruct((5, 2, NP, HH), F32)],
        mesh=plsc.VectorSubcoreMesh(core_axis_name="c", subcore_axis_name="s"),
        compiler_params=pltpu.CompilerParams(use_tc_tiling_on_sc=False),
        scratch_types=[
            pltpu.VMEM_SHARED((NP, HH), F32),
            pltpu.VMEM_SHARED((NP,), F32),
            pltpu.VMEM_SHARED((NP,), F32),
            pltpu.VMEM_SHARED((NP,), F32),
            pltpu.VMEM((CH,), jnp.int32),
            pltpu.VMEM((CH,), jnp.int32),
            pltpu.VMEM((CH,), jnp.int32),
            pltpu.VMEM((CH,), F32),
            pltpu.VMEM((CH,), F32),
            pltpu.VMEM((CH,), F32),
            pltpu.VMEM((CH, HH), F32),
            pltpu.VMEM((16,), F32),
            pltpu.SemaphoreType.DMA,
            pltpu.SemaphoreType.DMA,
            pltpu.SemaphoreType.DMA,
        ],
    )
    (out_edge,) = sc_edge(
        edge_co_occurs_with.reshape(2 * E), edge_used_in.reshape(2 * E),
        edge_pairs_with.reshape(2 * E), edge_follows.reshape(2 * E),
        edge_contains.reshape(2 * E), xic.reshape(2 * NP, HH),
        xdc.reshape(2 * NP, HH), svi, svd, a5, z2d, z1d)

    s1, tsum = pl.pallas_call(
        _post_body,
        grid=(NP // PB,),
        in_specs=[
            pl.BlockSpec((5, 1, PB, HH), lambda k: (0, 0, k, 0)),
            pl.BlockSpec((5, 1, PB, HH), lambda k: (0, 1, k, 0)),
            pl.BlockSpec((H, H), lambda k: (0, 0)),
            pl.BlockSpec((8, H), lambda k: (0, 0)),
        ],
        out_specs=[
            pl.BlockSpec((5, H), lambda k: (0, 0)),
            pl.BlockSpec((5, H), lambda k: (0, 0)),
        ],
        out_shape=[
            jax.ShapeDtypeStruct((5, H), F32),
            jax.ShapeDtypeStruct((5, H), F32),
        ],
    )(out_edge, out_edge, p["k_lin_W"], _b8(p["k_lin_b"], H))

    eps = jax.random.normal(jax.random.key(42), (1, H), dtype=F32)
    recon8, mu8, lv8 = pl.pallas_call(
        _tail_body,
        out_shape=[
            jax.ShapeDtypeStruct((8, F), F32),
            jax.ShapeDtypeStruct((8, H), F32),
            jax.ShapeDtypeStruct((8, H), F32),
        ],
    )(s1, tsum, _b8(p["q"], H), jnp.broadcast_to(cond[None, :], (8, C)),
      p["fc_mu_W"][0:H], p["fc_mu_W"][H:2 * H], p["fc_mu_W"][2 * H:],
      _b8(p["fc_mu_b"], H),
      p["fc_logvar_W"][0:H], p["fc_logvar_W"][H:2 * H],
      p["fc_logvar_W"][2 * H:], _b8(p["fc_logvar_b"], H),
      p["fc3_W"], _b8(p["fc3_b"], H), p["fc2_W"], _b8(p["fc2_b"], F),
      jnp.broadcast_to(eps, (8, H)))
    return recon8[0:1], mu8[0:1], lv8[0:1]


# speedup vs baseline: 52.3305x; 1.1571x over previous
import functools

import jax
import jax.numpy as jnp
from jax import lax
from jax.experimental import pallas as pl
from jax.experimental.pallas import tpu as pltpu
from jax.experimental.pallas import tpu_sc as plsc

N = 50000
NP = 51200
E = 800000
H = 64
HH = 32
F = 128
C = 16
CB = 2048
NBLK = NP // CB
PB = NP // 16
CH = 400
NCH = E // 16 // CH
SUBN = 400
NSUBN = PB // SUBN
BF16 = jnp.bfloat16
F32 = jnp.float32


def _prep_body(xi_ref, xd_ref, wi_ref, bi_ref, wd_ref, bd_ref, atti_ref,
               attd_ref, xio_ref, xdo_ref, svi_ref, svd_ref, ami_ref, amd_ref):
    k = pl.program_id(0)
    yi = jnp.dot(xi_ref[...], wi_ref[...],
                 preferred_element_type=F32) + bi_ref[0:1, :]
    yd = jnp.dot(xd_ref[...], wd_ref[...],
                 preferred_element_type=F32) + bd_ref[0:1, :]
    xio_ref[0] = yi[:, :HH]
    xio_ref[1] = yi[:, HH:]
    xdo_ref[0] = yd[:, :HH]
    xdo_ref[1] = yd[:, HH:]
    svi = lax.dot_general(atti_ref[...], yi, (((1,), (1,)), ((), ())),
                          preferred_element_type=F32)
    svd = lax.dot_general(attd_ref[...], yd, (((1,), (1,)), ((), ())),
                          preferred_element_type=F32)
    svi_ref[...] = svi
    svd_ref[...] = svd
    valid = (lax.broadcasted_iota(jnp.int32, (1, CB), 1) + k * CB) < N
    svi_m = jnp.where(valid, svi, -3.0e38)
    svd_m = jnp.where(valid, svd, -3.0e38)
    mi = jnp.broadcast_to(jnp.max(svi_m, axis=1, keepdims=True), (8, 128))
    md = jnp.broadcast_to(jnp.max(svd_m, axis=1, keepdims=True), (8, 128))

    @pl.when(k == 0)
    def _():
        ami_ref[...] = mi
        amd_ref[...] = md

    @pl.when(k != 0)
    def _():
        ami_ref[...] = jnp.maximum(ami_ref[...], mi)
        amd_ref[...] = jnp.maximum(amd_ref[...], md)


def _sc_body(e0, e1, e2, e3, e4, xic, xdc, svi, svd, a5, z2d, z1d,
             out_edge,
             acc, den, s_tab, d_tab,
             jv, iva, ivb, jv2, sg, dg, wv, rows, av,
             sem_e, sem_s, sem_d, sem_r, sem_w):
    cid = lax.axis_index("c")
    sid = lax.axis_index("s")
    segbase = sid * PB
    xoff = cid * NP

    es = [e0, e1, e2, e3, e4]
    plan = [(svi, 0, svd, 0, xic),
            (svi, 1, svd, 1, xic),
            (svd, 2, svd, 3, xdc),
            (svd, 4, svd, 5, xdc),
            (svd, 6, svi, 2, xdc)]

    for r in range(5):
        s_ref, s_row, d_ref, d_row, x_ref = plan[r]
        e_ref = es[r]
        ebase = sid * (E // 16)
        pltpu.sync_copy(s_ref.at[s_row, pl.ds(segbase, PB)],
                        s_tab.at[pl.ds(segbase, PB)])
        pltpu.sync_copy(d_ref.at[d_row, pl.ds(segbase, PB)],
                        d_tab.at[pl.ds(segbase, PB)])
        pltpu.sync_copy(z2d, acc.at[pl.ds(segbase, PB)])
        pltpu.sync_copy(z1d, den.at[pl.ds(segbase, PB)])
        pltpu.sync_copy(a5.at[pl.ds(r * 16, 16)], av)
        plsc.subcore_barrier()

        def chunk(c, carry):
            base = ebase + c * CH
            ce1 = pltpu.async_copy(e_ref.at[pl.ds(base, CH)], jv, sem_e)
            ce2 = pltpu.async_copy(e_ref.at[pl.ds(E + base, CH)], iva, sem_e)
            ce1.wait()
            ce2.wait()
            cp_s = pltpu.async_copy(s_tab.at[jv], sg, sem_s)
            cp_d = pltpu.async_copy(d_tab.at[iva], dg, sem_d)

            def adj(g, _):
                sl = pl.ds(g * 16, 16)
                jv2[sl] = jv[sl] + xoff
                return 0
            lax.fori_loop(0, CH // 16, adj, 0)
            cp_r = pltpu.async_copy(x_ref.at[jv2], rows, sem_r)
            cp_s.wait()
            cp_d.wait()
            avv = av[...]

            def wloop(g, _):
                sl = pl.ds(g * 16, 16)
                t = sg[sl] + dg[sl]
                al = jnp.maximum(t, 0.2 * t)
                bb = avv + dg[sl]
                b = jnp.maximum(bb, 0.2 * bb)
                wv[sl] = jnp.exp(al - b)
                return 0
            lax.fori_loop(0, CH // 16, wloop, 0)
            cp_r.wait()

            def mloop(g, _):
                w16 = wv[pl.ds(g * 16, 16)]
                for lane in range(16):
                    e = g * 16 + lane
                    w = w16[lane]
                    rows[e, pl.ds(0, 16)] = rows[e, pl.ds(0, 16)] * w
                    rows[e, pl.ds(16, 16)] = rows[e, pl.ds(16, 16)] * w
                return 0
            lax.fori_loop(0, CH // 16, mloop, 0)
            c1 = pltpu.async_copy(rows, acc.at[iva], sem_w, add=True)
            c2 = pltpu.async_copy(wv, den.at[iva], sem_w, add=True)
            c1.wait()
            c2.wait()
            return carry
        lax.fori_loop(0, NCH, chunk, 0)
        plsc.subcore_barrier()

        for sub in range(NSUBN):
            off = segbase + sub * SUBN
            pltpu.sync_copy(acc.at[pl.ds(off, SUBN)], rows.at[pl.ds(0, SUBN)])
            pltpu.sync_copy(den.at[pl.ds(off, SUBN)], sg.at[pl.ds(0, SUBN)])

            def dloop(g, _):
                inv16 = 1.0 / (sg[pl.ds(g * 16, 16)] + 1e-16)
                for lane in range(16):
                    e = g * 16 + lane
                    inv = inv16[lane]
                    rows[e, pl.ds(0, 16)] = jnp.maximum(
                        rows[e, pl.ds(0, 16)] * inv, 0.0)
                    rows[e, pl.ds(16, 16)] = jnp.maximum(
                        rows[e, pl.ds(16, 16)] * inv, 0.0)
                return 0
            lax.fori_loop(0, SUBN // 16, dloop, 0)
            pltpu.sync_copy(rows.at[pl.ds(0, SUBN)],
                            out_edge.at[r, cid, pl.ds(off, SUBN)])


def _post_body(lo_ref, hi_ref, kw_ref, kb_ref, s1_ref, t_ref):
    k = pl.program_id(0)
    ids = lax.broadcasted_iota(jnp.int32, (PB, 1), 0) + k * PB
    mask = ids < N
    cs = []
    ct = []
    for r in range(5):
        al = lo_ref[r, 0].astype(F32)
        ah = hi_ref[r, 0].astype(F32)
        s1 = jnp.concatenate([jnp.sum(al, axis=0), jnp.sum(ah, axis=0)])
        pre = (jnp.dot(al, kw_ref[:HH, :], preferred_element_type=F32)
               + jnp.dot(ah, kw_ref[HH:, :], preferred_element_type=F32)
               + kb_ref[0:1, :])
        th = jnp.where(mask, jnp.tanh(pre), 0.0)
        cs.append(s1[None, :])
        ct.append(jnp.sum(th, axis=0)[None, :])
    cs = jnp.concatenate(cs, axis=0)
    ct = jnp.concatenate(ct, axis=0)

    @pl.when(k == 0)
    def _():
        s1_ref[...] = cs
        t_ref[...] = ct

    @pl.when(k != 0)
    def _():
        s1_ref[...] = s1_ref[...] + cs
        t_ref[...] = t_ref[...] + ct


def _tail_body(s1_ref, t_ref, q_ref, cond_ref, wmua_ref, wmub_ref, wmuc_ref,
               bmu_ref, wlva_ref, wlvb_ref, wlvc_ref, blv_ref, w3_ref, b3_ref,
               w2_ref, b2_ref, eps_ref, recon_ref, mu_ref, lv_ref):
    s1 = s1_ref[...]
    kmat = t_ref[...] * (1.0 / N)
    scores = jnp.sum(kmat * q_ref[0:1, :], axis=1, keepdims=True)
    sc4 = scores[0:4]
    m = jnp.max(sc4)
    e = jnp.exp(sc4 - m)
    attn = e / jnp.sum(e)
    mean_dir = jnp.sum(attn * s1[0:4], axis=0, keepdims=True) * (1.0 / N)
    mean_ing = s1[4:5] * (1.0 / N)
    mi8 = jnp.broadcast_to(mean_ing, (8, H))
    md8 = jnp.broadcast_to(mean_dir, (8, H))
    cond = cond_ref[...]

    def head(wa, wb, wc, b):
        return (jnp.dot(mi8, wa[...], preferred_element_type=F32)
                + jnp.dot(md8, wb[...], preferred_element_type=F32)
                + jnp.dot(cond, wc[...], preferred_element_type=F32)
                + b[0:1, :])
    mu = head(wmua_ref, wmub_ref, wmuc_ref, bmu_ref)
    lv = head(wlva_ref, wlvb_ref, wlvc_ref, blv_ref)
    std = jnp.exp(0.5 * lv)
    z = mu + eps_ref[...] * std
    h = jnp.maximum(jnp.dot(z, w3_ref[...], preferred_element_type=F32)
                    + b3_ref[0:1, :], 0.0)
    recon = jnp.tanh(jnp.dot(h, w2_ref[...], preferred_element_type=F32)
                     + b2_ref[0:1, :])
    recon_ref[...] = recon
    mu_ref[...] = mu
    lv_ref[...] = lv


def _b8(v, w):
    return jnp.broadcast_to(v[None, :], (8, w)).astype(F32)


def kernel(x_ingredient, x_direction, cond, edge_co_occurs_with, edge_used_in,
           edge_contains, edge_pairs_with, edge_follows, params):
    p = params
    atti = jnp.stack([p["att_src_co_occurs_with"], p["att_src_used_in"],
                      p["att_dst_contains"]]
                     + [jnp.zeros((H,), F32)] * 5)
    attd = jnp.stack([p["att_dst_co_occurs_with"], p["att_dst_used_in"],
                      p["att_src_pairs_with"], p["att_dst_pairs_with"],
                      p["att_src_follows"], p["att_dst_follows"],
                      p["att_src_contains"], jnp.zeros((H,), F32)])

    xic, xdc, svi, svd, ami, amd = pl.pallas_call(
        _prep_body,
        grid=(NBLK,),
        in_specs=[
            pl.BlockSpec((CB, F), lambda k: (k, 0)),
            pl.BlockSpec((CB, F), lambda k: (k, 0)),
            pl.BlockSpec((F, H), lambda k: (0, 0)),
            pl.BlockSpec((8, H), lambda k: (0, 0)),
            pl.BlockSpec((F, H), lambda k: (0, 0)),
            pl.BlockSpec((8, H), lambda k: (0, 0)),
            pl.BlockSpec((8, H), lambda k: (0, 0)),
            pl.BlockSpec((8, H), lambda k: (0, 0)),
        ],
        out_specs=[
            pl.BlockSpec((2, CB, HH), lambda k: (0, k, 0)),
            pl.BlockSpec((2, CB, HH), lambda k: (0, k, 0)),
            pl.BlockSpec((8, CB), lambda k: (0, k)),
            pl.BlockSpec((8, CB), lambda k: (0, k)),
            pl.BlockSpec((8, 128), lambda k: (0, 0)),
            pl.BlockSpec((8, 128), lambda k: (0, 0)),
        ],
        out_shape=[
            jax.ShapeDtypeStruct((2, NP, HH), F32),
            jax.ShapeDtypeStruct((2, NP, HH), F32),
            jax.ShapeDtypeStruct((8, NP), F32),
            jax.ShapeDtypeStruct((8, NP), F32),
            jax.ShapeDtypeStruct((8, 128), F32),
            jax.ShapeDtypeStruct((8, 128), F32),
        ],
    )(x_ingredient, x_direction, p["proj_W_ingredient"],
      _b8(p["proj_b_ingredient"], H), p["proj_W_direction"],
      _b8(p["proj_b_direction"], H), atti, attd)

    a5 = jnp.broadcast_to(
        jnp.stack([ami[0, 0], ami[1, 0], amd[2, 0], amd[4, 0],
                   amd[6, 0]])[:, None], (5, 16)).reshape(80)
    z2d = jnp.zeros((PB, HH), F32)
    z1d = jnp.zeros((PB,), F32)

    sc_edge = pl.kernel(
        _sc_body,
        out_type=[jax.ShapeDtypeStruct((5, 2, NP, HH), F32)],
        mesh=plsc.VectorSubcoreMesh(core_axis_name="c", subcore_axis_name="s"),
        compiler_params=pltpu.CompilerParams(use_tc_tiling_on_sc=False),
        scratch_types=[
            pltpu.VMEM_SHARED((NP, HH), F32),
            pltpu.VMEM_SHARED((NP,), F32),
            pltpu.VMEM_SHARED((NP,), F32),
            pltpu.VMEM_SHARED((NP,), F32),
            pltpu.VMEM((CH,), jnp.int32),
            pltpu.VMEM((CH,), jnp.int32),
            pltpu.VMEM((CH,), jnp.int32),
            pltpu.VMEM((CH,), jnp.int32),
            pltpu.VMEM((CH,), F32),
            pltpu.VMEM((CH,), F32),
            pltpu.VMEM((CH,), F32),
            pltpu.VMEM((CH, HH), F32),
            pltpu.VMEM((16,), F32),
            pltpu.SemaphoreType.DMA,
            pltpu.SemaphoreType.DMA,
            pltpu.SemaphoreType.DMA,
            pltpu.SemaphoreType.DMA,
            pltpu.SemaphoreType.DMA,
        ],
    )
    (out_edge,) = sc_edge(
        edge_co_occurs_with.reshape(2 * E), edge_used_in.reshape(2 * E),
        edge_pairs_with.reshape(2 * E), edge_follows.reshape(2 * E),
        edge_contains.reshape(2 * E), xic.reshape(2 * NP, HH),
        xdc.reshape(2 * NP, HH), svi, svd, a5, z2d, z1d)

    s1, tsum = pl.pallas_call(
        _post_body,
        grid=(NP // PB,),
        in_specs=[
            pl.BlockSpec((5, 1, PB, HH), lambda k: (0, 0, k, 0)),
            pl.BlockSpec((5, 1, PB, HH), lambda k: (0, 1, k, 0)),
            pl.BlockSpec((H, H), lambda k: (0, 0)),
            pl.BlockSpec((8, H), lambda k: (0, 0)),
        ],
        out_specs=[
            pl.BlockSpec((5, H), lambda k: (0, 0)),
            pl.BlockSpec((5, H), lambda k: (0, 0)),
        ],
        out_shape=[
            jax.ShapeDtypeStruct((5, H), F32),
            jax.ShapeDtypeStruct((5, H), F32),
        ],
    )(out_edge, out_edge, p["k_lin_W"], _b8(p["k_lin_b"], H))

    eps = jax.random.normal(jax.random.key(42), (1, H), dtype=F32)
    recon8, mu8, lv8 = pl.pallas_call(
        _tail_body,
        out_shape=[
            jax.ShapeDtypeStruct((8, F), F32),
            jax.ShapeDtypeStruct((8, H), F32),
            jax.ShapeDtypeStruct((8, H), F32),
        ],
    )(s1, tsum, _b8(p["q"], H), jnp.broadcast_to(cond[None, :], (8, C)),
      p["fc_mu_W"][0:H], p["fc_mu_W"][H:2 * H], p["fc_mu_W"][2 * H:],
      _b8(p["fc_mu_b"], H),
      p["fc_logvar_W"][0:H], p["fc_logvar_W"][H:2 * H],
      p["fc_logvar_W"][2 * H:], _b8(p["fc_logvar_b"], H),
      p["fc3_W"], _b8(p["fc3_b"], H), p["fc2_W"], _b8(p["fc2_b"], F),
      jnp.broadcast_to(eps, (8, H)))
    return recon8[0:1], mu8[0:1], lv8[0:1]


# speedup vs baseline: 52.3701x; 1.0008x over previous
import jax
import jax.numpy as jnp
from jax import lax
from jax.experimental import pallas as pl
from jax.experimental.pallas import tpu as pltpu
from jax.experimental.pallas import tpu_sc as plsc

N = 50000
NP = 51200
E = 800000
H = 64
HH = 32
F = 128
C = 16
CB = 2048
NBLK = NP // CB
PB = NP // 16
CH = 400
NCH = E // 16 // CH
SUBN = 400
NSUBN = PB // SUBN
BF16 = jnp.bfloat16
F32 = jnp.float32


def _prep_body(xi_ref, xd_ref, wi_ref, bi_ref, wd_ref, bd_ref, atti_ref,
               attd_ref, xio_ref, xdo_ref, svi_ref, svd_ref, ami_ref, amd_ref):
    k = pl.program_id(0)
    yi = jnp.dot(xi_ref[...], wi_ref[...],
                 preferred_element_type=F32) + bi_ref[0:1, :]
    yd = jnp.dot(xd_ref[...], wd_ref[...],
                 preferred_element_type=F32) + bd_ref[0:1, :]
    xio_ref[0] = yi[:, :HH]
    xio_ref[1] = yi[:, HH:]
    xdo_ref[0] = yd[:, :HH]
    xdo_ref[1] = yd[:, HH:]
    svi = lax.dot_general(atti_ref[...], yi, (((1,), (1,)), ((), ())),
                          preferred_element_type=F32)
    svd = lax.dot_general(attd_ref[...], yd, (((1,), (1,)), ((), ())),
                          preferred_element_type=F32)
    svi_ref[...] = svi
    svd_ref[...] = svd
    valid = (lax.broadcasted_iota(jnp.int32, (1, CB), 1) + k * CB) < N
    svi_m = jnp.where(valid, svi, -3.0e38)
    svd_m = jnp.where(valid, svd, -3.0e38)
    mi = jnp.broadcast_to(jnp.max(svi_m, axis=1, keepdims=True), (8, 128))
    md = jnp.broadcast_to(jnp.max(svd_m, axis=1, keepdims=True), (8, 128))

    @pl.when(k == 0)
    def _():
        ami_ref[...] = mi
        amd_ref[...] = md

    @pl.when(k != 0)
    def _():
        ami_ref[...] = jnp.maximum(ami_ref[...], mi)
        amd_ref[...] = jnp.maximum(amd_ref[...], md)


def _sc_body(e0, e1, e2, e3, e4, xic, xdc, svi, svd, a5, z2d, z1d,
             out_edge,
             acc, den, s_tab, d_tab,
             jv, iva, ivb, jv2, sg, dg, wv, rows, av,
             sem_e, sem_s, sem_d, sem_r, sem_w):
    cid = lax.axis_index("c")
    sid = lax.axis_index("s")
    segbase = sid * PB
    xoff = cid * NP

    es = [e0, e1, e2, e3, e4]
    plan = [(svi, 0, svd, 0, xic),
            (svi, 1, svd, 1, xic),
            (svd, 2, svd, 3, xdc),
            (svd, 4, svd, 5, xdc),
            (svd, 6, svi, 2, xdc)]

    for r in range(5):
        s_ref, s_row, d_ref, d_row, x_ref = plan[r]
        e_ref = es[r]
        ebase = sid * (E // 16)
        pltpu.sync_copy(s_ref.at[s_row, pl.ds(segbase, PB)],
                        s_tab.at[pl.ds(segbase, PB)])
        pltpu.sync_copy(d_ref.at[d_row, pl.ds(segbase, PB)],
                        d_tab.at[pl.ds(segbase, PB)])
        pltpu.sync_copy(z2d, acc.at[pl.ds(segbase, PB)])
        pltpu.sync_copy(z1d, den.at[pl.ds(segbase, PB)])
        pltpu.sync_copy(a5.at[pl.ds(r * 16, 16)], av)
        plsc.subcore_barrier()

        def chunk(c, carry):
            base = ebase + c * CH
            ce1 = pltpu.async_copy(e_ref.at[pl.ds(base, CH)], jv, sem_e)
            ce2 = pltpu.async_copy(e_ref.at[pl.ds(E + base, CH)], iva, sem_e)
            ce1.wait()
            ce2.wait()
            cp_s = pltpu.async_copy(s_tab.at[jv], sg, sem_s)
            cp_d = pltpu.async_copy(d_tab.at[iva], dg, sem_d)

            def adj(g, _):
                sl = pl.ds(g * 16, 16)
                jv2[sl] = jv[sl] + xoff
                return 0
            lax.fori_loop(0, CH // 16, adj, 0)
            cp_r = pltpu.async_copy(x_ref.at[jv2], rows, sem_r)
            cp_s.wait()
            cp_d.wait()
            avv = av[...]

            def wloop(g, _):
                sl = pl.ds(g * 16, 16)
                t = sg[sl] + dg[sl]
                al = jnp.maximum(t, 0.2 * t)
                bb = avv + dg[sl]
                b = jnp.maximum(bb, 0.2 * bb)
                wv[sl] = jnp.exp(al - b)
                return 0
            lax.fori_loop(0, CH // 16, wloop, 0)
            cp_r.wait()

            def mloop(g, _):
                w16 = wv[pl.ds(g * 16, 16)]
                for lane in range(16):
                    e = g * 16 + lane
                    w = w16[lane]
                    rows[e, pl.ds(0, 16)] = rows[e, pl.ds(0, 16)] * w
                    rows[e, pl.ds(16, 16)] = rows[e, pl.ds(16, 16)] * w
                return 0
            lax.fori_loop(0, CH // 16, mloop, 0)
            c1 = pltpu.async_copy(rows, acc.at[iva], sem_w, add=True)
            c2 = pltpu.async_copy(wv, den.at[iva], sem_w, add=True)
            c1.wait()
            c2.wait()
            return carry
        lax.fori_loop(0, NCH, chunk, 0)
        plsc.subcore_barrier()

        for sub in range(NSUBN):
            off = segbase + sub * SUBN
            pltpu.sync_copy(acc.at[pl.ds(off, SUBN)], rows.at[pl.ds(0, SUBN)])
            pltpu.sync_copy(den.at[pl.ds(off, SUBN)], sg.at[pl.ds(0, SUBN)])

            def dloop(g, _):
                inv16 = 1.0 / (sg[pl.ds(g * 16, 16)] + 1e-16)
                for lane in range(16):
                    e = g * 16 + lane
                    inv = inv16[lane]
                    rows[e, pl.ds(0, 16)] = jnp.maximum(
                        rows[e, pl.ds(0, 16)] * inv, 0.0)
                    rows[e, pl.ds(16, 16)] = jnp.maximum(
                        rows[e, pl.ds(16, 16)] * inv, 0.0)
                return 0
            lax.fori_loop(0, SUBN // 16, dloop, 0)
            pltpu.sync_copy(rows.at[pl.ds(0, SUBN)],
                            out_edge.at[r, cid, pl.ds(off, SUBN)])


def _post_body(lo_ref, hi_ref, kw_ref, kb_ref, s1_ref, t_ref):
    k = pl.program_id(0)
    ids = lax.broadcasted_iota(jnp.int32, (PB, 1), 0) + k * PB
    mask = ids < N
    cs = []
    ct = []
    for r in range(5):
        al = lo_ref[r, 0].astype(F32)
        ah = hi_ref[r, 0].astype(F32)
        s1 = jnp.concatenate([jnp.sum(al, axis=0), jnp.sum(ah, axis=0)])
        pre = (jnp.dot(al, kw_ref[:HH, :], preferred_element_type=F32)
               + jnp.dot(ah, kw_ref[HH:, :], preferred_element_type=F32)
               + kb_ref[0:1, :])
        th = jnp.where(mask, jnp.tanh(pre), 0.0)
        cs.append(s1[None, :])
        ct.append(jnp.sum(th, axis=0)[None, :])
    cs = jnp.concatenate(cs, axis=0)
    ct = jnp.concatenate(ct, axis=0)

    @pl.when(k == 0)
    def _():
        s1_ref[...] = cs
        t_ref[...] = ct

    @pl.when(k != 0)
    def _():
        s1_ref[...] = s1_ref[...] + cs
        t_ref[...] = t_ref[...] + ct


def _tail_body(s1_ref, t_ref, q_ref, cond_ref, wmua_ref, wmub_ref, wmuc_ref,
               bmu_ref, wlva_ref, wlvb_ref, wlvc_ref, blv_ref, w3_ref, b3_ref,
               w2_ref, b2_ref, eps_ref, recon_ref, mu_ref, lv_ref):
    s1 = s1_ref[...]
    kmat = t_ref[...] * (1.0 / N)
    scores = jnp.sum(kmat * q_ref[0:1, :], axis=1, keepdims=True)
    sc4 = scores[0:4]
    m = jnp.max(sc4)
    e = jnp.exp(sc4 - m)
    attn = e / jnp.sum(e)
    mean_dir = jnp.sum(attn * s1[0:4], axis=0, keepdims=True) * (1.0 / N)
    mean_ing = s1[4:5] * (1.0 / N)
    mi8 = jnp.broadcast_to(mean_ing, (8, H))
    md8 = jnp.broadcast_to(mean_dir, (8, H))
    cond = cond_ref[...]

    def head(wa, wb, wc, b):
        return (jnp.dot(mi8, wa[...], preferred_element_type=F32)
                + jnp.dot(md8, wb[...], preferred_element_type=F32)
                + jnp.dot(cond, wc[...], preferred_element_type=F32)
                + b[0:1, :])
    mu = head(wmua_ref, wmub_ref, wmuc_ref, bmu_ref)
    lv = head(wlva_ref, wlvb_ref, wlvc_ref, blv_ref)
    std = jnp.exp(0.5 * lv)
    z = mu + eps_ref[...] * std
    h = jnp.maximum(jnp.dot(z, w3_ref[...], preferred_element_type=F32)
                    + b3_ref[0:1, :], 0.0)
    recon = jnp.tanh(jnp.dot(h, w2_ref[...], preferred_element_type=F32)
                     + b2_ref[0:1, :])
    recon_ref[...] = recon
    mu_ref[...] = mu
    lv_ref[...] = lv


def _b8(v, w):
    return jnp.broadcast_to(v[None, :], (8, w)).astype(F32)


def kernel(x_ingredient, x_direction, cond, edge_co_occurs_with, edge_used_in,
           edge_contains, edge_pairs_with, edge_follows, params):
    p = params
    atti = jnp.stack([p["att_src_co_occurs_with"], p["att_src_used_in"],
                      p["att_dst_contains"]]
                     + [jnp.zeros((H,), F32)] * 5)
    attd = jnp.stack([p["att_dst_co_occurs_with"], p["att_dst_used_in"],
                      p["att_src_pairs_with"], p["att_dst_pairs_with"],
                      p["att_src_follows"], p["att_dst_follows"],
                      p["att_src_contains"], jnp.zeros((H,), F32)])

    xic, xdc, svi, svd, ami, amd = pl.pallas_call(
        _prep_body,
        grid=(NBLK,),
        in_specs=[
            pl.BlockSpec((CB, F), lambda k: (k, 0)),
            pl.BlockSpec((CB, F), lambda k: (k, 0)),
            pl.BlockSpec((F, H), lambda k: (0, 0)),
            pl.BlockSpec((8, H), lambda k: (0, 0)),
            pl.BlockSpec((F, H), lambda k: (0, 0)),
            pl.BlockSpec((8, H), lambda k: (0, 0)),
            pl.BlockSpec((8, H), lambda k: (0, 0)),
            pl.BlockSpec((8, H), lambda k: (0, 0)),
        ],
        out_specs=[
            pl.BlockSpec((2, CB, HH), lambda k: (0, k, 0)),
            pl.BlockSpec((2, CB, HH), lambda k: (0, k, 0)),
            pl.BlockSpec((8, CB), lambda k: (0, k)),
            pl.BlockSpec((8, CB), lambda k: (0, k)),
            pl.BlockSpec((8, 128), lambda k: (0, 0)),
            pl.BlockSpec((8, 128), lambda k: (0, 0)),
        ],
        out_shape=[
            jax.ShapeDtypeStruct((2, NP, HH), F32),
            jax.ShapeDtypeStruct((2, NP, HH), F32),
            jax.ShapeDtypeStruct((8, NP), F32),
            jax.ShapeDtypeStruct((8, NP), F32),
            jax.ShapeDtypeStruct((8, 128), F32),
            jax.ShapeDtypeStruct((8, 128), F32),
        ],
    )(x_ingredient, x_direction, p["proj_W_ingredient"],
      _b8(p["proj_b_ingredient"], H), p["proj_W_direction"],
      _b8(p["proj_b_direction"], H), atti, attd)

    a5 = jnp.broadcast_to(
        jnp.stack([ami[0, 0], ami[1, 0], amd[2, 0], amd[4, 0],
                   amd[6, 0]])[:, None], (5, 16)).reshape(80)
    z2d = jnp.zeros((PB, HH), F32)
    z1d = jnp.zeros((PB,), F32)

    sc_edge = pl.kernel(
        _sc_body,
        out_type=[jax.ShapeDtypeStruct((5, 2, NP, HH), F32)],
        mesh=plsc.VectorSubcoreMesh(core_axis_name="c", subcore_axis_name="s"),
        compiler_params=pltpu.CompilerParams(use_tc_tiling_on_sc=False),
        scratch_types=[
            pltpu.VMEM_SHARED((NP, HH), F32),
            pltpu.VMEM_SHARED((NP,), F32),
            pltpu.VMEM_SHARED((NP,), F32),
            pltpu.VMEM_SHARED((NP,), F32),
            pltpu.VMEM((CH,), jnp.int32),
            pltpu.VMEM((CH,), jnp.int32),
            pltpu.VMEM((CH,), jnp.int32),
            pltpu.VMEM((CH,), jnp.int32),
            pltpu.VMEM((CH,), F32),
            pltpu.VMEM((CH,), F32),
            pltpu.VMEM((CH,), F32),
            pltpu.VMEM((CH, HH), F32),
            pltpu.VMEM((16,), F32),
            pltpu.SemaphoreType.DMA,
            pltpu.SemaphoreType.DMA,
            pltpu.SemaphoreType.DMA,
            pltpu.SemaphoreType.DMA,
            pltpu.SemaphoreType.DMA,
        ],
    )
    (out_edge,) = sc_edge(
        edge_co_occurs_with.reshape(2 * E), edge_used_in.reshape(2 * E),
        edge_pairs_with.reshape(2 * E), edge_follows.reshape(2 * E),
        edge_contains.reshape(2 * E), xic.reshape(2 * NP, HH),
        xdc.reshape(2 * NP, HH), svi, svd, a5, z2d, z1d)

    s1, tsum = pl.pallas_call(
        _post_body,
        grid=(NP // PB,),
        in_specs=[
            pl.BlockSpec((5, 1, PB, HH), lambda k: (0, 0, k, 0)),
            pl.BlockSpec((5, 1, PB, HH), lambda k: (0, 1, k, 0)),
            pl.BlockSpec((H, H), lambda k: (0, 0)),
            pl.BlockSpec((8, H), lambda k: (0, 0)),
        ],
        out_specs=[
            pl.BlockSpec((5, H), lambda k: (0, 0)),
            pl.BlockSpec((5, H), lambda k: (0, 0)),
        ],
        out_shape=[
            jax.ShapeDtypeStruct((5, H), F32),
            jax.ShapeDtypeStruct((5, H), F32),
        ],
    )(out_edge, out_edge, p["k_lin_W"], _b8(p["k_lin_b"], H))

    eps = jax.random.normal(jax.random.key(42), (1, H), dtype=F32)
    recon8, mu8, lv8 = pl.pallas_call(
        _tail_body,
        out_shape=[
            jax.ShapeDtypeStruct((8, F), F32),
            jax.ShapeDtypeStruct((8, H), F32),
            jax.ShapeDtypeStruct((8, H), F32),
        ],
    )(s1, tsum, _b8(p["q"], H), jnp.broadcast_to(cond[None, :], (8, C)),
      p["fc_mu_W"][0:H], p["fc_mu_W"][H:2 * H], p["fc_mu_W"][2 * H:],
      _b8(p["fc_mu_b"], H),
      p["fc_logvar_W"][0:H], p["fc_logvar_W"][H:2 * H],
      p["fc_logvar_W"][2 * H:], _b8(p["fc_logvar_b"], H),
      p["fc3_W"], _b8(p["fc3_b"], H), p["fc2_W"], _b8(p["fc2_b"], F),
      jnp.broadcast_to(eps, (8, H)))
    return recon8[0:1], mu8[0:1], lv8[0:1]
